# baseline (speedup 1.0000x reference)
import functools

import jax
import jax.numpy as jnp
from jax import lax
from jax.experimental import pallas as pl
from jax.experimental.pallas import tpu as pltpu
from jax.experimental.pallas import tpu_sc as plsc

LENGTH = 1000000
FEAT_DIM = 64
NEG_NUM = 4096
BATCH = 128
D_IN = 2048
MOMENTUM = 0.5
TEMPERATURE = 0.07

NC = 2
NS = 16
NW = NC * NS
BPW = BATCH // NW
CHUNK = 128
NCH = NEG_NUM // CHUNK


def _neck_body(x_ref, w_ref, o_ref):
    f = jnp.dot(x_ref[...], w_ref[...], preferred_element_type=jnp.float32)
    n = jnp.sqrt(jnp.sum(f * f, axis=1, keepdims=True))
    o_ref[...] = f / (n + 1e-12)


def _neck(x, w):
    return pl.pallas_call(
        _neck_body,
        out_shape=jax.ShapeDtypeStruct((BATCH, FEAT_DIM), jnp.float32),
    )(x, w)


def _sc_body(bank, negidx, posidx, feat, neg_out, pos_out,
             idxbuf, featv, buf0, buf1, part, logitbuf, pidx, pbuf,
             sem0, sem1, psem):
    cid = lax.axis_index("c")
    sid = lax.axis_index("s")
    wid = sid * NC + cid
    iot = lax.iota(jnp.int32, 16)

    @pl.when(wid < 8)
    def _():
        pltpu.sync_copy(posidx.at[wid], pidx)
        pltpu.async_copy(bank.at[pidx], pbuf, psem).wait()
        pltpu.sync_copy(pbuf, pos_out.at[pl.ds(wid * 16, 16)])

    def compute(buf, k, f0, f1, f2, f3):
        base_out = k * CHUNK

        def group(g, _):
            row0 = g * 16
            for j in range(16):
                r = row0 + j
                p = buf[r, pl.ds(0, 16)] * f0
                p = p + buf[r, pl.ds(16, 16)] * f1
                p = p + buf[r, pl.ds(32, 16)] * f2
                p = p + buf[r, pl.ds(48, 16)] * f3
                part[pl.ds(j * 16, 16)] = p
            acc = jnp.zeros((16,), jnp.float32)
            iot16 = iot * 16
            for cc in range(16):
                acc = acc + plsc.load_gather(part, [iot16 + cc])
            logitbuf[pl.ds(base_out + row0, 16)] = acc
            return 0

        lax.fori_loop(0, CHUNK // 16, group, 0)

    for bi in range(BPW):
        b = wid * BPW + bi
        pltpu.sync_copy(feat.at[b], featv)
        pltpu.sync_copy(negidx.at[pl.ds(b * NCH, NCH)], idxbuf)
        f0 = featv[pl.ds(0, 16)]
        f1 = featv[pl.ds(16, 16)]
        f2 = featv[pl.ds(32, 16)]
        f3 = featv[pl.ds(48, 16)]
        pltpu.async_copy(bank.at[idxbuf.at[0]], buf0, sem0)
        pltpu.async_copy(bank.at[idxbuf.at[1]], buf1, sem1)

        def pair(i, carry):
            k0 = 2 * i
            pltpu.make_async_copy(bank.at[idxbuf.at[k0]], buf0, sem0).wait()
            compute(buf0, k0, f0, f1, f2, f3)

            @pl.when(i < NCH // 2 - 1)
            def _():
                pltpu.async_copy(bank.at[idxbuf.at[k0 + 2]], buf0, sem0)

            k1 = 2 * i + 1
            pltpu.make_async_copy(bank.at[idxbuf.at[k1]], buf1, sem1).wait()
            compute(buf1, k1, f0, f1, f2, f3)

            @pl.when(i < NCH // 2 - 1)
            def _():
                pltpu.async_copy(bank.at[idxbuf.at[k1 + 2]], buf1, sem1)

            return carry

        lax.fori_loop(0, NCH // 2, pair, 0)
        pltpu.sync_copy(logitbuf, neg_out.at[b])


def _sc_gather_dot(bank, negidx2d, posidx, feature):
    mesh = plsc.VectorSubcoreMesh(core_axis_name="c", subcore_axis_name="s",
                                  num_cores=NC, num_subcores=NS)
    fn = pl.kernel(
        _sc_body,
        out_type=(
            jax.ShapeDtypeStruct((BATCH, NEG_NUM), jnp.float32),
            jax.ShapeDtypeStruct((BATCH, FEAT_DIM), jnp.float32),
        ),
        mesh=mesh,
        compiler_params=pltpu.CompilerParams(needs_layout_passes=False,
                                             use_tc_tiling_on_sc=False),
        scratch_types=[
            pltpu.VMEM((NCH, CHUNK), jnp.int32),
            pltpu.VMEM((FEAT_DIM,), jnp.float32),
            pltpu.VMEM((CHUNK, FEAT_DIM), jnp.float32),
            pltpu.VMEM((CHUNK, FEAT_DIM), jnp.float32),
            pltpu.VMEM((256,), jnp.float32),
            pltpu.VMEM((NEG_NUM,), jnp.float32),
            pltpu.VMEM((16,), jnp.int32),
            pltpu.VMEM((16, FEAT_DIM), jnp.float32),
            pltpu.SemaphoreType.DMA,
            pltpu.SemaphoreType.DMA,
            pltpu.SemaphoreType.DMA,
        ],
    )
    return fn(bank, negidx2d, posidx, feature)


def _head_body(feat_ref, pos_ref, neg_ref, loss_ref, new_ref):
    feat = feat_ref[...]
    posf = pos_ref[...]
    inv_t = 1.0 / TEMPERATURE
    pos_l = jnp.sum(posf * feat, axis=1, keepdims=True) * inv_t
    neg_l = neg_ref[...] * inv_t
    m = jnp.maximum(jnp.max(neg_l, axis=1, keepdims=True), pos_l)
    se = jnp.sum(jnp.exp(neg_l - m), axis=1, keepdims=True) + jnp.exp(pos_l - m)
    lse = m + jnp.log(se)
    loss_ref[...] = jnp.broadcast_to(-jnp.mean(pos_l - lse), (1, 1))
    new = (1.0 - MOMENTUM) * posf + MOMENTUM * feat
    nn = jnp.sqrt(jnp.sum(new * new, axis=1, keepdims=True))
    new_ref[...] = new / (nn + 1e-12)


def _head(feature, pos_feat, neg_logits):
    return pl.pallas_call(
        _head_body,
        out_shape=(
            jax.ShapeDtypeStruct((1, 1), jnp.float32),
            jax.ShapeDtypeStruct((BATCH, FEAT_DIM), jnp.float32),
        ),
    )(feature, pos_feat, neg_logits)


def _scatter_body(idx_ref, new_ref, bank_ref, out_ref):
    del idx_ref, bank_ref
    out_ref[...] = new_ref[...]


def _scatter(idx, new_rows, bank):
    out = pl.pallas_call(
        _scatter_body,
        grid_spec=pltpu.PrefetchScalarGridSpec(
            num_scalar_prefetch=1,
            grid=(BATCH,),
            in_specs=[
                pl.BlockSpec((1, 1, FEAT_DIM), lambda i, idx_ref: (i, 0, 0)),
                pl.BlockSpec(memory_space=pl.ANY),
            ],
            out_specs=pl.BlockSpec(
                (1, 1, FEAT_DIM), lambda i, idx_ref: (idx_ref[i], 0, 0)),
        ),
        out_shape=jax.ShapeDtypeStruct((LENGTH, 1, FEAT_DIM), jnp.float32),
        input_output_aliases={2: 0},
    )(idx, new_rows.reshape(BATCH, 1, FEAT_DIM),
      bank.reshape(LENGTH, 1, FEAT_DIM))
    return out.reshape(LENGTH, FEAT_DIM)


def kernel(feature_in, neck_W, feature_bank, idx, neg_idx):
    feature = _neck(feature_in, neck_W)
    negidx2d = neg_idx.reshape(NEG_NUM // CHUNK * BATCH, CHUNK)
    posidx = idx.reshape(8, 16)
    neg_logits, pos_feat = _sc_gather_dot(feature_bank, negidx2d, posidx,
                                          feature)
    loss11, new_rows = _head(feature, pos_feat, neg_logits)
    new_bank = _scatter(idx, new_rows, feature_bank)
    return loss11[0, 0], new_bank
